# baseline (speedup 1.0000x reference)
import jax
import jax.numpy as jnp
import numpy as np
from jax import lax
from jax.experimental import pallas as pl
from jax.experimental.pallas import tpu as pltpu
from jax.experimental.pallas import tpu_sc as plsc

_NFREQ = 3
_MU = 11.0
_B = 4096
_S = 200
_N = _B * _S
_NWORKERS = 32
_EPW = _N // _NWORKERS
_E = 1280
_NCHUNKS = _EPW // _E
_PA = 39
_PB = 38


def _fourier_table_t() -> np.ndarray:
    b = np.arange(256)
    xi = ((b >> 4) & 15).astype(np.float32)
    yi = (b & 15).astype(np.float32)
    xn = xi / np.float32(_MU - 1.0) * np.float32(2.0) - np.float32(1.0)
    yn = yi / np.float32(_MU - 1.0) * np.float32(2.0) - np.float32(1.0)
    freqs = (2.0 ** np.arange(_NFREQ)).astype(np.float32)
    xs = xn[:, None] * freqs[None, :]
    ys = yn[:, None] * freqs[None, :]
    t = np.zeros((256, 12), dtype=np.float32)
    t[:, 0:3] = np.cos(xs)
    t[:, 3:6] = np.sin(xs)
    t[:, 6:9] = np.cos(ys)
    t[:, 9:12] = np.sin(ys)
    return np.ascontiguousarray(t.T)


_TXYT = _fourier_table_t()


def _sc_body(td_hbm, wt_hbm, txyt_hbm, out_hbm, td_v, out_a, out_b,
             wt_v, txyt_v, sem_in0, sem_in1, sem_a, sem_b):
    wid = lax.axis_index("s") * 2 + lax.axis_index("c")
    base = wid * _EPW
    sems_in = (sem_in0, sem_in1)
    pltpu.sync_copy(wt_hbm, wt_v)
    pltpu.sync_copy(txyt_hbm, txyt_v)
    pltpu.async_copy(td_hbm.at[:, pl.ds(base, _E)], td_v.at[0], sems_in[0])

    def pair(tp, _):
        for b in range(2):
            t = 2 * tp + b
            n0 = base + t * _E
            pltpu.make_async_copy(
                td_hbm.at[:, pl.ds(n0, _E)], td_v.at[b], sems_in[b]).wait()

            @pl.when(t + 1 < _NCHUNKS)
            def _():
                pltpu.async_copy(td_hbm.at[:, pl.ds(n0 + _E, _E)],
                                 td_v.at[1 - b], sems_in[1 - b])

            @pl.when(t >= 1)
            def _():
                pltpu.make_async_copy(
                    out_a, out_hbm.at[pl.ds(0, _PA), pl.ds(n0 - _E, _E)],
                    sem_a).wait()
                pltpu.make_async_copy(
                    out_b, out_hbm.at[pl.ds(_PA, _PB), pl.ds(n0 - _E, _E)],
                    sem_b).wait()

            @plsc.parallel_loop(0, _E, step=16, unroll=2)
            def grp(o):
                attr = td_v[b, 1, pl.ds(o, 16)] & 255
                byte = td_v[b, 0, pl.ds(o, 16)] & 255
                for p0 in range(0, 64, 8):
                    vals = [plsc.load_gather(
                                wt_v,
                                [jnp.full((16,), p0 + i, jnp.int32), attr])
                            for i in range(8)]
                    for i in range(8):
                        if p0 + i < _PA:
                            out_a[p0 + i, pl.ds(o, 16)] = vals[i]
                        else:
                            out_b[p0 + i - _PA, pl.ds(o, 16)] = vals[i]
                fvals = [plsc.load_gather(
                             txyt_v, [jnp.full((16,), p, jnp.int32), byte])
                         for p in range(12)]
                for p in range(12):
                    out_b[64 - _PA + p, pl.ds(o, 16)] = fvals[p]
                out_b[76 - _PA, pl.ds(o, 16)] = (
                    td_v[b, 2, pl.ds(o, 16)].astype(jnp.float32))

            pltpu.async_copy(
                out_a, out_hbm.at[pl.ds(0, _PA), pl.ds(n0, _E)], sem_a)
            pltpu.async_copy(
                out_b, out_hbm.at[pl.ds(_PA, _PB), pl.ds(n0, _E)], sem_b)
        return 0

    lax.fori_loop(0, _NCHUNKS // 2, pair, 0)
    end = base + (_NCHUNKS - 1) * _E
    pltpu.make_async_copy(
        out_a, out_hbm.at[pl.ds(0, _PA), pl.ds(end, _E)], sem_a).wait()
    pltpu.make_async_copy(
        out_b, out_hbm.at[pl.ds(_PA, _PB), pl.ds(end, _E)], sem_b).wait()


@jax.jit
def _run(td_lin, wt, txyt):
    mesh = plsc.VectorSubcoreMesh(core_axis_name="c", subcore_axis_name="s")
    f = pl.kernel(
        _sc_body,
        out_type=jax.ShapeDtypeStruct((77, _N), jnp.float32),
        mesh=mesh,
        scratch_types=[
            pltpu.VMEM((2, 3, _E), jnp.int32),
            pltpu.VMEM((_PA, _E), jnp.float32),
            pltpu.VMEM((_PB, _E), jnp.float32),
            pltpu.VMEM((64, 256), jnp.float32),
            pltpu.VMEM((12, 256), jnp.float32),
            pltpu.SemaphoreType.DMA,
            pltpu.SemaphoreType.DMA,
            pltpu.SemaphoreType.DMA,
            pltpu.SemaphoreType.DMA,
        ],
        compiler_params=pltpu.CompilerParams(
            use_tc_tiling_on_sc=False, needs_layout_passes=False),
    )
    return f(td_lin, wt, txyt)


def kernel(td, W):
    td_lin = (td.transpose(2, 1, 0)
                .reshape(3, _S // 8, 8, _B // 128, 128)
                .transpose(0, 1, 3, 2, 4)
                .reshape(3, _N))
    k5 = _run(td_lin, W.T, jnp.asarray(_TXYT))
    out = (k5.reshape(77, _S // 8, _B // 128, 8, 128)
             .transpose(2, 4, 1, 3, 0)
             .reshape(_B, _S, 77))
    return out


# speedup vs baseline: 1.3526x; 1.3526x over previous
import jax
import jax.numpy as jnp
import numpy as np
from jax import lax
from jax.experimental import pallas as pl
from jax.experimental.pallas import tpu as pltpu
from jax.experimental.pallas import tpu_sc as plsc

_NFREQ = 3
_MU = 11.0
_B = 4096
_S = 200
_N = _B * _S
_NWORKERS = 32
_EPW = _N // _NWORKERS
_E = 1280
_NCHUNKS = _EPW // _E
_PA = 39
_PB = 38


def _fourier_table_t() -> np.ndarray:
    b = np.arange(256)
    xi = ((b >> 4) & 15).astype(np.float32)
    yi = (b & 15).astype(np.float32)
    xn = xi / np.float32(_MU - 1.0) * np.float32(2.0) - np.float32(1.0)
    yn = yi / np.float32(_MU - 1.0) * np.float32(2.0) - np.float32(1.0)
    freqs = (2.0 ** np.arange(_NFREQ)).astype(np.float32)
    xs = xn[:, None] * freqs[None, :]
    ys = yn[:, None] * freqs[None, :]
    t = np.zeros((256, 12), dtype=np.float32)
    t[:, 0:3] = np.cos(xs)
    t[:, 3:6] = np.sin(xs)
    t[:, 6:9] = np.cos(ys)
    t[:, 9:12] = np.sin(ys)
    return np.ascontiguousarray(t.T)


_TXYT = _fourier_table_t()


def _sc_body(td_hbm, wt_hbm, txyt_hbm, out_hbm, td_v, out_a, out_b,
             wt_v, txyt_v, sem_in0, sem_in1, sem_a, sem_b):
    wid = lax.axis_index("s") * 2 + lax.axis_index("c")
    base = wid * _EPW
    sems_in = (sem_in0, sem_in1)
    pltpu.sync_copy(wt_hbm, wt_v)
    pltpu.sync_copy(txyt_hbm, txyt_v)
    pltpu.async_copy(td_hbm.at[:, pl.ds(base, _E)], td_v.at[0], sems_in[0])

    def pair(tp, _):
        for b in range(2):
            t = 2 * tp + b
            n0 = base + t * _E
            pltpu.make_async_copy(
                td_hbm.at[:, pl.ds(n0, _E)], td_v.at[b], sems_in[b]).wait()

            @pl.when(t + 1 < _NCHUNKS)
            def _():
                pltpu.async_copy(td_hbm.at[:, pl.ds(n0 + _E, _E)],
                                 td_v.at[1 - b], sems_in[1 - b])

            @pl.when(t >= 1)
            def _():
                pltpu.make_async_copy(
                    out_a, out_hbm.at[pl.ds(0, _PA), pl.ds(n0 - _E, _E)],
                    sem_a).wait()

            @plsc.parallel_loop(0, _E, step=16, unroll=2)
            def grp_a(o):
                attr = td_v[b, 1, pl.ds(o, 16)] & 255
                for p0 in range(0, _PA, 8):
                    k = min(8, _PA - p0)
                    vals = [plsc.load_gather(
                                wt_v,
                                [jnp.full((16,), p0 + i, jnp.int32), attr])
                            for i in range(k)]
                    for i in range(k):
                        out_a[p0 + i, pl.ds(o, 16)] = vals[i]

            pltpu.async_copy(
                out_a, out_hbm.at[pl.ds(0, _PA), pl.ds(n0, _E)], sem_a)

            @pl.when(t >= 1)
            def _():
                pltpu.make_async_copy(
                    out_b, out_hbm.at[pl.ds(_PA, _PB), pl.ds(n0 - _E, _E)],
                    sem_b).wait()

            @plsc.parallel_loop(0, _E, step=16, unroll=2)
            def grp_b(o):
                attr = td_v[b, 1, pl.ds(o, 16)] & 255
                byte = td_v[b, 0, pl.ds(o, 16)] & 255
                for p0 in range(_PA, 64, 8):
                    k = min(8, 64 - p0)
                    vals = [plsc.load_gather(
                                wt_v,
                                [jnp.full((16,), p0 + i, jnp.int32), attr])
                            for i in range(k)]
                    for i in range(k):
                        out_b[p0 - _PA + i, pl.ds(o, 16)] = vals[i]
                fvals = [plsc.load_gather(
                             txyt_v, [jnp.full((16,), p, jnp.int32), byte])
                         for p in range(12)]
                for p in range(12):
                    out_b[64 - _PA + p, pl.ds(o, 16)] = fvals[p]
                out_b[76 - _PA, pl.ds(o, 16)] = (
                    td_v[b, 2, pl.ds(o, 16)].astype(jnp.float32))

            pltpu.async_copy(
                out_b, out_hbm.at[pl.ds(_PA, _PB), pl.ds(n0, _E)], sem_b)
        return 0

    lax.fori_loop(0, _NCHUNKS // 2, pair, 0)
    end = base + (_NCHUNKS - 1) * _E
    pltpu.make_async_copy(
        out_a, out_hbm.at[pl.ds(0, _PA), pl.ds(end, _E)], sem_a).wait()
    pltpu.make_async_copy(
        out_b, out_hbm.at[pl.ds(_PA, _PB), pl.ds(end, _E)], sem_b).wait()


@jax.jit
def _run(td_lin, wt, txyt):
    mesh = plsc.VectorSubcoreMesh(core_axis_name="c", subcore_axis_name="s")
    f = pl.kernel(
        _sc_body,
        out_type=jax.ShapeDtypeStruct((77, _N), jnp.float32),
        mesh=mesh,
        scratch_types=[
            pltpu.VMEM((2, 3, _E), jnp.int32),
            pltpu.VMEM((_PA, _E), jnp.float32),
            pltpu.VMEM((_PB, _E), jnp.float32),
            pltpu.VMEM((64, 256), jnp.float32),
            pltpu.VMEM((12, 256), jnp.float32),
            pltpu.SemaphoreType.DMA,
            pltpu.SemaphoreType.DMA,
            pltpu.SemaphoreType.DMA,
            pltpu.SemaphoreType.DMA,
        ],
        compiler_params=pltpu.CompilerParams(
            use_tc_tiling_on_sc=False, needs_layout_passes=False),
    )
    return f(td_lin, wt, txyt)


def kernel(td, W):
    td_lin = (td.transpose(2, 1, 0)
                .reshape(3, _S // 8, 8, _B // 128, 128)
                .transpose(0, 1, 3, 2, 4)
                .reshape(3, _N))
    k5 = _run(td_lin, W.T, jnp.asarray(_TXYT))
    out = (k5.reshape(77, _S // 8, _B // 128, 8, 128)
             .transpose(2, 4, 1, 3, 0)
             .reshape(_B, _S, 77))
    return out
